# Initial kernel scaffold; baseline (speedup 1.0000x reference)
import functools
import jax
import jax.numpy as jnp
from jax import lax
from jax.experimental import pallas as pl
from jax.experimental.pallas import tpu as pltpu
from jax.experimental.pallas import tpu_sc as plsc

D = 64
HALF = 25000
HALF_PAD = 25024
TRASH = HALF_PAD
ACC_ROWS = HALF_PAD + 8
NP = 2 * HALF_PAD
N = 2 * HALF
E = 800000
C = 128
CHUNKS = 391
EPT = C * CHUNKS
EP = 16 * EPT
ROWS_PT = HALF_PAD // 16
R = 92
RCH = ROWS_PT // R


def _spmm_body(scale, xp, x0p, src, dst, w, out,
               acc, srcv, dstv, wv, idxv, rows, cbuf, bbuf, sem):
  c = lax.axis_index("c")
  s = lax.axis_index("s")
  row0 = s * ROWS_PT
  half0 = c * HALF

  def zrow(r, carry):
    for j in range(4):
      cbuf[r, pl.ds(16 * j, 16)] = jnp.zeros((16,), jnp.float32)
    return carry
  lax.fori_loop(0, R, zrow, 0)

  def zchunk(k, carry):
    pltpu.sync_copy(cbuf, acc.at[pl.ds(row0 + k * R, R)])
    return carry
  lax.fori_loop(0, RCH, zchunk, 0)
  plsc.subcore_barrier()

  e0 = s * EPT

  def echunk(k, carry):
    base = e0 + k * C
    pltpu.sync_copy(src.at[pl.ds(base, C)], srcv)
    pltpu.sync_copy(dst.at[pl.ds(base, C)], dstv)
    pltpu.sync_copy(w.at[pl.ds(base, C)], wv)

    def idxg(g, carry2):
      sl = pl.ds(g * 16, 16)
      sv = srcv[sl]
      srcv[sl] = sv + jnp.where(sv >= HALF, 24, 0).astype(jnp.int32)
      dv = dstv[sl] - half0
      ok = (dv >= 0) & (dv < HALF)
      idxv[sl] = jnp.where(ok, dv, TRASH).astype(jnp.int32)
      return carry2
    lax.fori_loop(0, C // 16, idxg, 0)

    pltpu.async_copy(xp.at[srcv], rows, sem).wait()

    def sgrp(g, carry2):
      for r in range(16):
        e = g * 16 + r
        ws = plsc.load_gather(wv, [jnp.full((16,), e, jnp.int32)])
        for j in range(4):
          sl = pl.ds(16 * j, 16)
          rows[e, sl] = rows[e, sl] * ws
      return carry2
    lax.fori_loop(0, C // 16, sgrp, 0)

    pltpu.sync_copy(rows, acc.at[idxv], add=True)
    return carry
  lax.fori_loop(0, CHUNKS, echunk, 0)
  plsc.subcore_barrier()

  def ochunk(k, carry):
    r0 = row0 + k * R
    g0 = c * HALF_PAD + r0
    pltpu.sync_copy(acc.at[pl.ds(r0, R)], cbuf)
    pltpu.sync_copy(x0p.at[pl.ds(g0, R)], bbuf)

    def orow(r, carry2):
      for j in range(4):
        sl = pl.ds(16 * j, 16)
        cbuf[r, sl] = (cbuf[r, sl] + bbuf[r, sl]) * scale
      return carry2
    lax.fori_loop(0, R, orow, 0)
    pltpu.sync_copy(cbuf, out.at[pl.ds(g0, R)])
    return carry
  lax.fori_loop(0, RCH, ochunk, 0)


def _make_spmm(scale):
  return pl.kernel(
      functools.partial(_spmm_body, scale),
      out_type=jax.ShapeDtypeStruct((NP, D), jnp.float32),
      mesh=plsc.VectorSubcoreMesh(core_axis_name="c", subcore_axis_name="s"),
      scratch_types=[
          pltpu.VMEM_SHARED((ACC_ROWS, D), jnp.float32),
          pltpu.VMEM((C,), jnp.int32),
          pltpu.VMEM((C,), jnp.int32),
          pltpu.VMEM((C,), jnp.float32),
          pltpu.VMEM((C,), jnp.int32),
          pltpu.VMEM((C, D), jnp.float32),
          pltpu.VMEM((R, D), jnp.float32),
          pltpu.VMEM((R, D), jnp.float32),
          pltpu.SemaphoreType.DMA,
      ],
  )


_spmm_layer1 = _make_spmm(1.0)
_spmm_layer2 = _make_spmm(1.0 / 3.0)


@jax.jit
def kernel(sym_emb, herb_emb, edge_index, edge_weight):
  pad = jnp.zeros((HALF_PAD - HALF, D), jnp.float32)
  x0p = jnp.concatenate([sym_emb, pad, herb_emb, pad], axis=0)
  src = edge_index[0].astype(jnp.int32)
  dst = edge_index[1].astype(jnp.int32)
  epad = EP - E
  src = jnp.concatenate([src, jnp.zeros((epad,), jnp.int32)])
  dst = jnp.concatenate([dst, jnp.full((epad,), N, jnp.int32)])
  w = jnp.concatenate([edge_weight, jnp.zeros((epad,), jnp.float32)])
  s1 = _spmm_layer1(x0p, x0p, src, dst, w)
  out = _spmm_layer2(s1, x0p, src, dst, w)
  return out[:HALF], out[HALF_PAD:HALF_PAD + HALF]


# speedup vs baseline: 1.9581x; 1.9581x over previous
import functools
import jax
import jax.numpy as jnp
from jax import lax
from jax.experimental import pallas as pl
from jax.experimental.pallas import tpu as pltpu
from jax.experimental.pallas import tpu_sc as plsc

D = 64
HALF = 25000
HALF_PAD = 25088
TRASH = HALF_PAD
ACC_ROWS = HALF_PAD + 8
NP = 2 * HALF_PAD
N = 2 * HALF
E = 800000
C = 128
CHUNKS = 391
EPT = C * CHUNKS
EP = 16 * EPT
ROWS_PT = HALF_PAD // 16
R = 112
RCH = ROWS_PT // R


def _spmm_body(scale, xp, x0p, src, dst, w, out,
               acc, srcv, dstv, wv, idxv, rows, cbuf, bbuf, sem):
  c = lax.axis_index("c")
  s = lax.axis_index("s")
  row0 = s * ROWS_PT
  half0 = c * HALF

  def zrow(r, carry):
    for j in range(4):
      cbuf[r, pl.ds(16 * j, 16)] = jnp.zeros((16,), jnp.float32)
    return carry
  lax.fori_loop(0, R, zrow, 0)

  def zchunk(k, carry):
    pltpu.sync_copy(cbuf, acc.at[pl.ds(row0 + k * R, R)])
    return carry
  lax.fori_loop(0, RCH, zchunk, 0)
  plsc.subcore_barrier()

  e0 = s * EPT

  def echunk(k, carry):
    base = e0 + k * C
    pltpu.sync_copy(src.at[pl.ds(base, C)], srcv)
    pltpu.sync_copy(dst.at[pl.ds(base, C)], dstv)
    pltpu.sync_copy(w.at[pl.ds(base, C)], wv)

    def idxg(g, carry2):
      sl = pl.ds(g * 16, 16)
      sv = srcv[sl]
      srcv[sl] = sv + jnp.where(sv >= HALF, HALF_PAD - HALF, 0).astype(jnp.int32)
      dv = dstv[sl] - half0
      ok = (dv >= 0) & (dv < HALF)
      idxv[sl] = jnp.where(ok, dv, TRASH).astype(jnp.int32)
      return carry2
    lax.fori_loop(0, C // 16, idxg, 0)

    pltpu.async_copy(xp.at[srcv], rows, sem).wait()

    def sgrp(g, carry2):
      for r in range(16):
        e = g * 16 + r
        ws = plsc.load_gather(wv, [jnp.full((16,), e, jnp.int32)])
        for j in range(4):
          sl = pl.ds(16 * j, 16)
          rows[e, sl] = rows[e, sl] * ws
      return carry2
    lax.fori_loop(0, C // 16, sgrp, 0)

    pltpu.sync_copy(rows, acc.at[idxv], add=True)
    return carry
  lax.fori_loop(0, CHUNKS, echunk, 0)
  plsc.subcore_barrier()

  def ochunk(k, carry):
    r0 = row0 + k * R
    g0 = c * HALF_PAD + r0
    pltpu.sync_copy(acc.at[pl.ds(r0, R)], cbuf)
    pltpu.sync_copy(x0p.at[pl.ds(g0, R)], bbuf)

    def orow(r, carry2):
      for j in range(4):
        sl = pl.ds(16 * j, 16)
        cbuf[r, sl] = (cbuf[r, sl] + bbuf[r, sl]) * scale
      return carry2
    lax.fori_loop(0, R, orow, 0)
    pltpu.sync_copy(cbuf, out.at[pl.ds(g0, R)])
    return carry
  lax.fori_loop(0, RCH, ochunk, 0)


def _make_spmm(scale):
  return pl.kernel(
      functools.partial(_spmm_body, scale),
      out_type=jax.ShapeDtypeStruct((NP, D), jnp.float32),
      mesh=plsc.VectorSubcoreMesh(core_axis_name="c", subcore_axis_name="s"),
      compiler_params=pltpu.CompilerParams(
          needs_layout_passes=False, use_tc_tiling_on_sc=False),
      scratch_types=[
          pltpu.VMEM_SHARED((ACC_ROWS, D), jnp.float32),
          pltpu.VMEM((C,), jnp.int32),
          pltpu.VMEM((C,), jnp.int32),
          pltpu.VMEM((C,), jnp.float32),
          pltpu.VMEM((C,), jnp.int32),
          pltpu.VMEM((C, D), jnp.float32),
          pltpu.VMEM((R, D), jnp.float32),
          pltpu.VMEM((R, D), jnp.float32),
          pltpu.SemaphoreType.DMA,
      ],
  )


_spmm_layer1 = _make_spmm(1.0)
_spmm_layer2 = _make_spmm(1.0 / 3.0)


@jax.jit
def kernel(sym_emb, herb_emb, edge_index, edge_weight):
  pad = jnp.zeros((HALF_PAD - HALF, D), jnp.float32)
  x0p = jnp.concatenate([sym_emb, pad, herb_emb, pad], axis=0)
  src = edge_index[0].astype(jnp.int32)
  dst = edge_index[1].astype(jnp.int32)
  epad = EP - E
  src = jnp.concatenate([src, jnp.zeros((epad,), jnp.int32)])
  dst = jnp.concatenate([dst, jnp.full((epad,), N, jnp.int32)])
  w = jnp.concatenate([edge_weight, jnp.zeros((epad,), jnp.float32)])
  s1 = _spmm_layer1(x0p, x0p, src, dst, w)
  out = _spmm_layer2(s1, x0p, src, dst, w)
  return out[:HALF], out[HALF_PAD:HALF_PAD + HALF]


# speedup vs baseline: 2.3608x; 1.2057x over previous
import functools
import jax
import jax.numpy as jnp
from jax import lax
from jax.experimental import pallas as pl
from jax.experimental.pallas import tpu as pltpu
from jax.experimental.pallas import tpu_sc as plsc

D = 64
HALF = 25000
HALF_PAD = 25088
TRASH = HALF_PAD
ACC_ROWS = HALF_PAD + 8
NP = 2 * HALF_PAD
N = 2 * HALF
E = 800000
C = 128
CHUNKS = 392
EPT = C * CHUNKS
EP = 16 * EPT
ROWS_PT = HALF_PAD // 16
R = 112
RCH = ROWS_PT // R


def _spmm_body(scale, xp, x0p, src, dst, w, out,
               acc, srcv0, dstv0, wv0, idxv0, rows0,
               srcv1, dstv1, wv1, idxv1, rows1,
               gsem0, gsem1, ssem0, ssem1):
  cbuf = rows0
  bbuf = rows1
  srcv = (srcv0, srcv1)
  dstv = (dstv0, dstv1)
  wv = (wv0, wv1)
  idxv = (idxv0, idxv1)
  rows = (rows0, rows1)
  gsem = (gsem0, gsem1)
  ssem = (ssem0, ssem1)

  c = lax.axis_index("c")
  s = lax.axis_index("s")
  row0 = s * ROWS_PT
  half0 = c * HALF

  def zrow(r, carry):
    for j in range(4):
      cbuf[r, pl.ds(16 * j, 16)] = jnp.zeros((16,), jnp.float32)
    return carry
  lax.fori_loop(0, R, zrow, 0)

  def zchunk(k, carry):
    pltpu.sync_copy(cbuf.at[pl.ds(0, R)], acc.at[pl.ds(row0 + k * R, R)])
    return carry
  lax.fori_loop(0, RCH, zchunk, 0)
  plsc.subcore_barrier()

  e0 = s * EPT

  def prep(k, b):
    base = e0 + k * C
    pltpu.sync_copy(src.at[pl.ds(base, C)], srcv[b])
    pltpu.sync_copy(dst.at[pl.ds(base, C)], dstv[b])
    pltpu.sync_copy(w.at[pl.ds(base, C)], wv[b])

    def idxg(g, carry):
      sl = pl.ds(g * 16, 16)
      sv = srcv[b][sl]
      srcv[b][sl] = sv + jnp.where(sv >= HALF, HALF_PAD - HALF, 0).astype(
          jnp.int32)
      dv = dstv[b][sl] - half0
      ok = (dv >= 0) & (dv < HALF)
      idxv[b][sl] = jnp.where(ok, dv, TRASH).astype(jnp.int32)
      return carry
    lax.fori_loop(0, C // 16, idxg, 0)
    pltpu.async_copy(xp.at[srcv[b]], rows[b], gsem[b])

  prep(0, 0)

  def echunk2(i, carry):
    for b in (0, 1):
      k = 2 * i + b
      ob = 1 - b

      @pl.when(k + 1 < CHUNKS)
      def _():
        @pl.when(k >= 1)
        def _():
          pltpu.make_async_copy(rows[ob], acc.at[pl.ds(0, C)], ssem[ob]).wait()
        prep(k + 1, ob)

      pltpu.make_async_copy(xp.at[pl.ds(0, C)], rows[b], gsem[b]).wait()

      def sgrp(g, carry2):
        for r in range(16):
          e = g * 16 + r
          ws = plsc.load_gather(wv[b], [jnp.full((16,), e, jnp.int32)])
          for j in range(4):
            sl = pl.ds(16 * j, 16)
            rows[b][e, sl] = rows[b][e, sl] * ws
        return carry2
      lax.fori_loop(0, C // 16, sgrp, 0)

      pltpu.async_copy(rows[b], acc.at[idxv[b]], ssem[b], add=True)
    return carry
  lax.fori_loop(0, CHUNKS // 2, echunk2, 0)
  pltpu.make_async_copy(rows[0], acc.at[pl.ds(0, C)], ssem[0]).wait()
  pltpu.make_async_copy(rows[1], acc.at[pl.ds(0, C)], ssem[1]).wait()
  plsc.subcore_barrier()

  def ochunk(k, carry):
    r0 = row0 + k * R
    g0 = c * HALF_PAD + r0
    pltpu.sync_copy(acc.at[pl.ds(r0, R)], cbuf.at[pl.ds(0, R)])
    pltpu.sync_copy(x0p.at[pl.ds(g0, R)], bbuf.at[pl.ds(0, R)])

    def orow(r, carry2):
      for j in range(4):
        sl = pl.ds(16 * j, 16)
        cbuf[r, sl] = (cbuf[r, sl] + bbuf[r, sl]) * scale
      return carry2
    lax.fori_loop(0, R, orow, 0)
    pltpu.sync_copy(cbuf.at[pl.ds(0, R)], out.at[pl.ds(g0, R)])
    return carry
  lax.fori_loop(0, RCH, ochunk, 0)


def _make_spmm(scale):
  return pl.kernel(
      functools.partial(_spmm_body, scale),
      out_type=jax.ShapeDtypeStruct((NP, D), jnp.float32),
      mesh=plsc.VectorSubcoreMesh(core_axis_name="c", subcore_axis_name="s"),
      compiler_params=pltpu.CompilerParams(
          needs_layout_passes=False, use_tc_tiling_on_sc=False),
      scratch_types=[
          pltpu.VMEM_SHARED((ACC_ROWS, D), jnp.float32),
          pltpu.VMEM((C,), jnp.int32),
          pltpu.VMEM((C,), jnp.int32),
          pltpu.VMEM((C,), jnp.float32),
          pltpu.VMEM((C,), jnp.int32),
          pltpu.VMEM((C, D), jnp.float32),
          pltpu.VMEM((C,), jnp.int32),
          pltpu.VMEM((C,), jnp.int32),
          pltpu.VMEM((C,), jnp.float32),
          pltpu.VMEM((C,), jnp.int32),
          pltpu.VMEM((C, D), jnp.float32),
          pltpu.SemaphoreType.DMA,
          pltpu.SemaphoreType.DMA,
          pltpu.SemaphoreType.DMA,
          pltpu.SemaphoreType.DMA,
      ],
  )


_spmm_layer1 = _make_spmm(1.0)
_spmm_layer2 = _make_spmm(1.0 / 3.0)


@jax.jit
def kernel(sym_emb, herb_emb, edge_index, edge_weight):
  pad = jnp.zeros((HALF_PAD - HALF, D), jnp.float32)
  x0p = jnp.concatenate([sym_emb, pad, herb_emb, pad], axis=0)
  src = edge_index[0].astype(jnp.int32)
  dst = edge_index[1].astype(jnp.int32)
  epad = EP - E
  src = jnp.concatenate([src, jnp.zeros((epad,), jnp.int32)])
  dst = jnp.concatenate([dst, jnp.full((epad,), N, jnp.int32)])
  w = jnp.concatenate([edge_weight, jnp.zeros((epad,), jnp.float32)])
  s1 = _spmm_layer1(x0p, x0p, src, dst, w)
  out = _spmm_layer2(s1, x0p, src, dst, w)
  return out[:HALF], out[HALF_PAD:HALF_PAD + HALF]


# speedup vs baseline: 3.0686x; 1.2998x over previous
import functools
import jax
import jax.numpy as jnp
from jax import lax
from jax.experimental import pallas as pl
from jax.experimental.pallas import tpu as pltpu
from jax.experimental.pallas import tpu_sc as plsc

D = 64
HALF = 25000
HALF_PAD = 25088
TRASH0 = HALF_PAD - 64
ACC_ROWS = HALF_PAD
NP = 2 * HALF_PAD
N = 2 * HALF
E = 800000
C = 96
SUP = 8
NSUP = 66
EPT = C * SUP * NSUP
EP = 16 * EPT
ER = EP // C
ROWS_PT = HALF_PAD // 16
R = 56
RCH = ROWS_PT // R


def _spmm_body(scale, xp, x0p, src, dst, w, out,
               acc, srcv0, srcv1, dstv0, dstv1, wv0, wv1,
               rows0, rows1, rows2, rows3,
               isem0, isem1, gsem0, gsem1, gsem2, gsem3,
               ssem0, ssem1, ssem2, ssem3):
  srcv = (srcv0, srcv1)
  dstv = (dstv0, dstv1)
  wv = (wv0, wv1)
  rows = (rows0, rows1, rows2, rows3)
  isem = (isem0, isem1)
  gsem = (gsem0, gsem1, gsem2, gsem3)
  ssem = (ssem0, ssem1, ssem2, ssem3)

  c = lax.axis_index("c")
  s = lax.axis_index("s")
  row0 = s * ROWS_PT
  half0 = c * HALF
  trash = TRASH0 + s

  zb = rows[0]

  def zrow(r, carry):
    for j in range(4):
      zb[r, pl.ds(16 * j, 16)] = jnp.zeros((16,), jnp.float32)
    return carry
  lax.fori_loop(0, R, zrow, 0)

  def zchunk(k, carry):
    pltpu.sync_copy(zb.at[pl.ds(0, R)], acc.at[pl.ds(row0 + k * R, R)])
    return carry
  lax.fori_loop(0, RCH, zchunk, 0)
  plsc.subcore_barrier()

  er0 = s * (SUP * NSUP)

  def idx_load(S, b):
    base = er0 + S * SUP
    pltpu.async_copy(src.at[pl.ds(base, SUP)], srcv[b], isem[b])
    pltpu.async_copy(dst.at[pl.ds(base, SUP)], dstv[b], isem[b])
    pltpu.async_copy(w.at[pl.ds(base, SUP)], wv[b], isem[b])

  def idx_wait(b):
    pltpu.make_async_copy(src.at[pl.ds(0, SUP)], srcv[b], isem[b]).wait()
    pltpu.make_async_copy(dst.at[pl.ds(0, SUP)], dstv[b], isem[b]).wait()
    pltpu.make_async_copy(w.at[pl.ds(0, SUP)], wv[b], isem[b]).wait()

  def transform(b):
    def trow(jr, carry):
      for g in range(C // 16):
        sl = pl.ds(g * 16, 16)
        sv = srcv[b][jr, sl]
        srcv[b][jr, sl] = sv + jnp.where(
            sv >= HALF, HALF_PAD - HALF, 0).astype(jnp.int32)
        dv = dstv[b][jr, sl] - half0
        ok = (dv >= 0) & (dv < HALF)
        dstv[b][jr, sl] = jnp.where(ok, dv, trash).astype(jnp.int32)
      return carry
    lax.fori_loop(0, SUP, trow, 0)

  def gather_issue(b, j, slot):
    pltpu.async_copy(xp.at[srcv[b].at[j]], rows[slot], gsem[slot])

  def gather_wait(slot):
    pltpu.make_async_copy(xp.at[pl.ds(0, C)], rows[slot], gsem[slot]).wait()

  def scatter_issue(b, j, slot):
    pltpu.async_copy(rows[slot], acc.at[dstv[b].at[j]], ssem[slot], add=True)

  def scatter_drain(slot):
    pltpu.make_async_copy(rows[slot], acc.at[pl.ds(0, C)], ssem[slot]).wait()

  def scale_chunk(b, j, slot):
    def sgrp(g, carry):
      for r in range(16):
        e = g * 16 + r
        ws = plsc.load_gather(
            wv[b], [jnp.full((16,), j, jnp.int32), jnp.full((16,), e, jnp.int32)])
        for q in range(4):
          sl = pl.ds(16 * q, 16)
          rows[slot][e, sl] = rows[slot][e, sl] * ws
      return carry
    lax.fori_loop(0, C // 16, sgrp, 0)

  idx_load(0, 0)
  idx_wait(0)
  transform(0)
  gather_issue(0, 0, 0)
  gather_issue(0, 1, 1)

  def super2(i, carry):
    for sb in (0, 1):
      S = 2 * i + sb
      nb = 1 - sb

      for j in range(SUP):
        slot = j % 4
        if j == 2:
          @pl.when(S + 1 < NSUP)
          def _():
            idx_load(S + 1, nb)
        gather_wait(slot)
        scale_chunk(sb, j, slot)
        scatter_issue(sb, j, slot)
        if j == 4:
          @pl.when(S + 1 < NSUP)
          def _():
            idx_wait(nb)
            transform(nb)
        nslot = (j + 2) % 4
        if j < SUP - 2:
          if j >= 2:
            scatter_drain(nslot)
          else:
            @pl.when(S >= 1)
            def _():
              scatter_drain(nslot)
          gather_issue(sb, j + 2, nslot)
        else:
          @pl.when(S + 1 < NSUP)
          def _():
            scatter_drain(nslot)
            gather_issue(nb, j + 2 - SUP, nslot)
    return carry
  lax.fori_loop(0, NSUP // 2, super2, 0)
  scatter_drain(0)
  scatter_drain(1)
  scatter_drain(2)
  scatter_drain(3)
  plsc.subcore_barrier()

  cbuf = rows[0]
  bbuf = rows[1]

  def ochunk(k, carry):
    r0 = row0 + k * R
    g0 = c * HALF_PAD + r0
    pltpu.sync_copy(acc.at[pl.ds(r0, R)], cbuf.at[pl.ds(0, R)])
    pltpu.sync_copy(x0p.at[pl.ds(g0, R)], bbuf.at[pl.ds(0, R)])

    def orow(r, carry2):
      for j in range(4):
        sl = pl.ds(16 * j, 16)
        cbuf[r, sl] = (cbuf[r, sl] + bbuf[r, sl]) * scale
      return carry2
    lax.fori_loop(0, R, orow, 0)
    pltpu.sync_copy(cbuf.at[pl.ds(0, R)], out.at[pl.ds(g0, R)])
    return carry
  lax.fori_loop(0, RCH, ochunk, 0)


def _make_spmm(scale):
  return pl.kernel(
      functools.partial(_spmm_body, scale),
      out_type=jax.ShapeDtypeStruct((NP, D), jnp.float32),
      mesh=plsc.VectorSubcoreMesh(core_axis_name="c", subcore_axis_name="s"),
      compiler_params=pltpu.CompilerParams(
          needs_layout_passes=False, use_tc_tiling_on_sc=False),
      scratch_types=[
          pltpu.VMEM_SHARED((ACC_ROWS, D), jnp.float32),
          pltpu.VMEM((SUP, C), jnp.int32),
          pltpu.VMEM((SUP, C), jnp.int32),
          pltpu.VMEM((SUP, C), jnp.int32),
          pltpu.VMEM((SUP, C), jnp.int32),
          pltpu.VMEM((SUP, C), jnp.float32),
          pltpu.VMEM((SUP, C), jnp.float32),
          pltpu.VMEM((C, D), jnp.float32),
          pltpu.VMEM((C, D), jnp.float32),
          pltpu.VMEM((C, D), jnp.float32),
          pltpu.VMEM((C, D), jnp.float32),
          pltpu.SemaphoreType.DMA,
          pltpu.SemaphoreType.DMA,
          pltpu.SemaphoreType.DMA,
          pltpu.SemaphoreType.DMA,
          pltpu.SemaphoreType.DMA,
          pltpu.SemaphoreType.DMA,
          pltpu.SemaphoreType.DMA,
          pltpu.SemaphoreType.DMA,
          pltpu.SemaphoreType.DMA,
          pltpu.SemaphoreType.DMA,
      ],
  )


_spmm_layer1 = _make_spmm(1.0)
_spmm_layer2 = _make_spmm(1.0 / 3.0)


@jax.jit
def kernel(sym_emb, herb_emb, edge_index, edge_weight):
  pad = jnp.zeros((HALF_PAD - HALF, D), jnp.float32)
  x0p = jnp.concatenate([sym_emb, pad, herb_emb, pad], axis=0)
  src = edge_index[0].astype(jnp.int32)
  dst = edge_index[1].astype(jnp.int32)
  epad = EP - E
  src = jnp.concatenate([src, jnp.zeros((epad,), jnp.int32)]).reshape(ER, C)
  dst = jnp.concatenate([dst, jnp.full((epad,), N, jnp.int32)]).reshape(ER, C)
  w = jnp.concatenate([edge_weight,
                       jnp.zeros((epad,), jnp.float32)]).reshape(ER, C)
  s1 = _spmm_layer1(x0p, x0p, src, dst, w)
  out = _spmm_layer2(s1, x0p, src, dst, w)
  return out[:HALF], out[HALF_PAD:HALF_PAD + HALF]
